# Initial kernel scaffold; baseline (speedup 1.0000x reference)
import jax
import jax.numpy as jnp
from jax.experimental import pallas as pl


def kernel(feat, edge_index, dist, W1, b1, W2, b2, Watt, belta):
    raise NotImplementedError("write your pallas kernel here")


# speedup vs baseline: 4.2450x; 4.2450x over previous
import functools

import jax
import jax.numpy as jnp
from jax import lax
from jax.experimental import pallas as pl
from jax.experimental.pallas import tpu as pltpu
from jax.experimental.pallas import tpu_sc as plsc

N_NODES = 10000
N_EDGES = 320000
D = 128

NC = 2
NS = 16
L = 16
NW = NC * NS
E_PER_W = N_EDGES // NW
CHUNK = 80
N_CHUNKS = E_PER_W // CHUNK
N_PAD = 10240
ROWS_PER_TILE = N_PAD // NS


def _leaky_relu(x):
    return jnp.where(x > 0, x, 0.2 * x)


def _fc1_body(feat_ref, w1t_ref, b1_ref, out_ref):
    out_ref[...] = _leaky_relu(
        jnp.dot(feat_ref[...], w1t_ref[...], preferred_element_type=jnp.float32)
        + b1_ref[...])


def _final_body(p_ref, feat_ref, w2t_ref, b2_ref, wattt_ref, out_ref):
    feat = feat_ref[...]
    ft = p_ref[0] + p_ref[1]
    e = jnp.dot(feat, wattt_ref[...], preferred_element_type=jnp.float32)
    rst = ft + e * feat
    out_ref[...] = _leaky_relu(
        jnp.dot(rst, w2t_ref[...], preferred_element_type=jnp.float32)
        + b2_ref[...])


_sc_mesh = plsc.VectorSubcoreMesh(core_axis_name="c", subcore_axis_name="s")


@functools.partial(
    pl.kernel,
    out_type=jax.ShapeDtypeStruct((NC, N_PAD, D), jnp.float32),
    mesh=_sc_mesh,
    scratch_types=[
        pltpu.VMEM((CHUNK,), jnp.int32),
        pltpu.VMEM((CHUNK,), jnp.int32),
        pltpu.VMEM((CHUNK,), jnp.float32),
        pltpu.VMEM((CHUNK, D), jnp.float32),
        pltpu.VMEM((L,), jnp.float32),
        pltpu.VMEM_SHARED((N_PAD, D), jnp.float32),
        pltpu.SemaphoreType.DMA,
    ],
)
def _segment_sum_sc(src_hbm, dst_hbm, dist_hbm, belta_hbm, zero_hbm, fs_hbm,
                    out_hbm, src_v, dst_v, dist_v, rows_v, belta_v,
                    ft_sh, sem):
    cid = lax.axis_index("c")
    sid = lax.axis_index("s")
    wid = cid * NS + sid

    r0 = sid * ROWS_PER_TILE
    pltpu.sync_copy(zero_hbm.at[pl.ds(r0, ROWS_PER_TILE)],
                    ft_sh.at[pl.ds(r0, ROWS_PER_TILE)])
    pltpu.sync_copy(belta_hbm, belta_v)
    plsc.subcore_barrier()

    bv = belta_v[...]
    base_e = wid * E_PER_W

    def chunk_body(k, carry):
        b = base_e + k * CHUNK
        pltpu.sync_copy(src_hbm.at[pl.ds(b, CHUNK)], src_v)
        pltpu.sync_copy(dst_hbm.at[pl.ds(b, CHUNK)], dst_v)
        pltpu.sync_copy(dist_hbm.at[pl.ds(b, CHUNK)], dist_v)
        pltpu.async_copy(fs_hbm.at[src_v], rows_v, sem).wait()
        for g in range(CHUNK // L):
            w16 = bv / dist_v[pl.ds(g * L, L)]
            for j in range(L):
                wj = jnp.full((L,), w16[j], jnp.float32)
                row = g * L + j
                for f in range(D // L):
                    sl = pl.ds(f * L, L)
                    rows_v[row, sl] = rows_v[row, sl] * wj
        pltpu.sync_copy(rows_v, ft_sh.at[dst_v], add=True)
        return carry

    lax.fori_loop(0, N_CHUNKS, chunk_body, 0)
    plsc.subcore_barrier()
    pltpu.sync_copy(ft_sh.at[pl.ds(r0, ROWS_PER_TILE)],
                    out_hbm.at[cid, pl.ds(r0, ROWS_PER_TILE)])


def kernel(feat, edge_index, dist, W1, b1, W2, b2, Watt, belta):
    src = edge_index[0]
    dst = edge_index[1]
    w1t = W1.T
    w2t = W2.T
    wattt = Watt.T
    b1r = b1.reshape(1, D)
    b2r = b2.reshape(1, D)
    belta_vec = jnp.broadcast_to(belta, (L,))
    zeros = jnp.zeros((N_PAD, D), jnp.float32)

    feat_src = pl.pallas_call(
        _fc1_body,
        out_shape=jax.ShapeDtypeStruct((N_NODES, D), jnp.float32),
    )(feat, w1t, b1r)

    partials = _segment_sum_sc(src, dst, dist, belta_vec, zeros, feat_src)
    partials = partials[:, :N_NODES, :]

    rst = pl.pallas_call(
        _final_body,
        out_shape=jax.ShapeDtypeStruct((N_NODES, D), jnp.float32),
    )(partials, feat, w2t, b2r, wattt)
    return rst
